# baseline (speedup 1.0000x reference)
import jax
import jax.numpy as jnp
from jax import lax
from jax.experimental import pallas as pl
from jax.experimental.pallas import tpu as pltpu
from jax.experimental.pallas import tpu_sc as plsc
import functools

NC = 2
NS = 16
L = 16
NW = NC * NS

B = 16384
F = 64
B_PER_W = B // NW


def _mf_kernel(uidx_hbm, iidx_hbm, uf_hbm, if_hbm, ub_hbm, ib_hbm, gb_hbm,
               out_hbm,
               uidx_v, iidx_v, urows_v, vrows_v, ubias_v, ibias_v, gb_v,
               out_v, acc_v, sem):
    wid = lax.axis_index("s") * NC + lax.axis_index("c")
    base = wid * B_PER_W

    pltpu.sync_copy(uidx_hbm.at[pl.ds(base, B_PER_W)], uidx_v)
    pltpu.sync_copy(iidx_hbm.at[pl.ds(base, B_PER_W)], iidx_v)

    c1 = pltpu.async_copy(uf_hbm.at[uidx_v], urows_v, sem)
    c2 = pltpu.async_copy(if_hbm.at[iidx_v], vrows_v, sem)
    c3 = pltpu.async_copy(ub_hbm.at[uidx_v], ubias_v, sem)
    c4 = pltpu.async_copy(ib_hbm.at[iidx_v], ibias_v, sem)
    pltpu.sync_copy(gb_hbm, gb_v.at[pl.ds(0, 1)])
    c1.wait()
    c2.wait()
    c3.wait()
    c4.wait()

    gb = gb_v[pl.ds(0, L)][0]
    lane = lax.iota(jnp.int32, 16)

    def body(g, _):
        gbase = g * L
        for j in range(L):
            b = gbase + j
            acc = urows_v[b, pl.ds(0, L)] * vrows_v[b, pl.ds(0, L)]
            for k in range(1, F // L):
                acc = acc + urows_v[b, pl.ds(k * L, L)] * vrows_v[b, pl.ds(k * L, L)]
            acc_v[j, pl.ds(0, L)] = acc
        dot = plsc.load_gather(acc_v, [lane, jnp.zeros((L,), jnp.int32)])
        for k in range(1, L):
            dot = dot + plsc.load_gather(acc_v, [lane, jnp.full((L,), k, jnp.int32)])
        out_v[pl.ds(gbase, L)] = (dot + ubias_v[pl.ds(gbase, L)]
                                  + ibias_v[pl.ds(gbase, L)] + gb)
        return 0

    lax.fori_loop(0, B_PER_W // L, body, 0)

    pltpu.sync_copy(out_v, out_hbm.at[pl.ds(base, B_PER_W)])


@jax.jit
def _run(user_idx, item_idx, user_factors, item_factors, ub_flat, ib_flat,
         global_bias):
    mesh = plsc.VectorSubcoreMesh(core_axis_name="c", subcore_axis_name="s",
                                  num_cores=NC, num_subcores=NS)
    return pl.kernel(
        _mf_kernel,
        out_type=jax.ShapeDtypeStruct((B,), jnp.float32),
        mesh=mesh,
        scratch_types=[
            pltpu.VMEM((B_PER_W,), jnp.int32),
            pltpu.VMEM((B_PER_W,), jnp.int32),
            pltpu.VMEM((B_PER_W, F), jnp.float32),
            pltpu.VMEM((B_PER_W, F), jnp.float32),
            pltpu.VMEM((B_PER_W,), jnp.float32),
            pltpu.VMEM((B_PER_W,), jnp.float32),
            pltpu.VMEM((L,), jnp.float32),
            pltpu.VMEM((B_PER_W,), jnp.float32),
            pltpu.VMEM((L, L + 1), jnp.float32),
            pltpu.SemaphoreType.DMA,
        ],
        compiler_params=pltpu.CompilerParams(needs_layout_passes=False,
                                             use_tc_tiling_on_sc=False),
    )(user_idx, item_idx, user_factors, item_factors, ub_flat, ib_flat,
      global_bias)


def kernel(user_idx, item_idx, user_factors, item_factors, user_bias,
           item_bias, global_bias):
    user_idx = user_idx.astype(jnp.int32)
    item_idx = item_idx.astype(jnp.int32)
    ub_flat = user_bias.reshape(-1)
    ib_flat = item_bias.reshape(-1)
    return _run(user_idx, item_idx, user_factors, item_factors, ub_flat,
                ib_flat, global_bias)
